# Initial kernel scaffold; baseline (speedup 1.0000x reference)
import jax
import jax.numpy as jnp
from jax.experimental import pallas as pl


def kernel(x, y, projections):
    raise NotImplementedError("write your pallas kernel here")


# speedup vs baseline: 1052.5623x; 1052.5623x over previous
import functools

import jax
import jax.numpy as jnp
from jax import lax
from jax.experimental import pallas as pl
from jax.experimental.pallas import tpu as pltpu
from jax.experimental.pallas import tpu_sc as plsc

_NUM_PROJ = 32
_EPS = 1e-06
_B, _T, _D = 4, 4096, 1024
_TB = 512
_ROWS = _B * _NUM_PROJ
_NW = 32
_PAIRS_PER_W = _ROWS // _NW
_NV = _T // 16


def _proj_body(x_ref, y_ref, p_ref, px_ref, py_ref):
    p = p_ref[...]
    pss = jnp.sum(p * p, axis=0, keepdims=True)
    pn = p / jnp.maximum(jnp.sqrt(pss), _EPS)
    for src, dst in ((x_ref, px_ref), (y_ref, py_ref)):
        v = src[0]
        ss = jnp.sum(v * v, axis=1)
        s = 1.0 / jnp.maximum(jnp.sqrt(ss), _EPS)
        out = lax.dot_general(
            pn, v, (((0,), (1,)), ((), ())),
            preferred_element_type=jnp.float32,
            precision=lax.Precision.HIGHEST)
        dst[...] = out * s[None, :]


def _project(x, y, projections):
    grid = (_B, _T // _TB)
    return pl.pallas_call(
        _proj_body,
        grid=grid,
        in_specs=[
            pl.BlockSpec((1, _TB, _D), lambda b, t: (b, t, 0)),
            pl.BlockSpec((1, _TB, _D), lambda b, t: (b, t, 0)),
            pl.BlockSpec((_D, _NUM_PROJ), lambda b, t: (0, 0)),
        ],
        out_specs=[
            pl.BlockSpec((_NUM_PROJ, _TB), lambda b, t: (b, t)),
            pl.BlockSpec((_NUM_PROJ, _TB), lambda b, t: (b, t)),
        ],
        out_shape=[
            jax.ShapeDtypeStruct((_ROWS, _T), jnp.float32),
            jax.ShapeDtypeStruct((_ROWS, _T), jnp.float32),
        ],
    )(x, y, projections)


def _sort16_pass(buf):
    def body(j, c):
        off = j * 16
        for r in range(8):
            buf[r, pl.ds(off, 16)] = jnp.sort(buf[r, pl.ds(off, 16)])
        return c
    lax.fori_loop(0, _NV, body, 0)


def _sc_cost_body(px_hbm, py_hbm, out_hbm, buf, costref):
    c = lax.axis_index("c")
    s = lax.axis_index("s")
    w = s * 2 + c
    base = w * _PAIRS_PER_W
    pltpu.sync_copy(px_hbm.at[pl.ds(base, _PAIRS_PER_W)],
                    buf.at[pl.ds(0, _PAIRS_PER_W)])
    pltpu.sync_copy(py_hbm.at[pl.ds(base, _PAIRS_PER_W)],
                    buf.at[pl.ds(_PAIRS_PER_W, _PAIRS_PER_W)])

    _sort16_pass(buf)

    for n in (16, 32, 64, 128, 256, 512, 1024, 2048):
        nb = n // 16

        def cross_body(p, c, nb=nb, n=n):
            blk = p // nb
            r16 = p % nb
            ai = blk * (2 * n) + r16 * 16
            bi = blk * (2 * n) + 2 * n - r16 * 16 - 16
            for r in range(8):
                av = buf[r, pl.ds(ai, 16)]
                bv = jnp.flip(buf[r, pl.ds(bi, 16)], axis=0)
                buf[r, pl.ds(ai, 16)] = jnp.minimum(av, bv)
                buf[r, pl.ds(bi, 16)] = jnp.flip(jnp.maximum(av, bv), axis=0)
            return c
        lax.fori_loop(0, _T // 32, cross_body, 0)

        d = n // 2
        while d >= 16:
            q = d // 16

            def stage_body(p, c, q=q, d=d):
                blk = p // q
                r16 = p % q
                i0 = blk * (2 * d) + r16 * 16
                i1 = i0 + d
                for r in range(8):
                    av = buf[r, pl.ds(i0, 16)]
                    bv = buf[r, pl.ds(i1, 16)]
                    buf[r, pl.ds(i0, 16)] = jnp.minimum(av, bv)
                    buf[r, pl.ds(i1, 16)] = jnp.maximum(av, bv)
                return c
            lax.fori_loop(0, _T // 32, stage_body, 0)
            d //= 2

        _sort16_pass(buf)

    lane = lax.iota(jnp.int32, 16)
    costv = jnp.zeros((16,), jnp.float32)
    inv_t = 1.0 / _T
    for i in range(_PAIRS_PER_W):
        def acc_body(j, acc, i=i):
            off = j * 16
            dlt = buf[i, pl.ds(off, 16)] - buf[_PAIRS_PER_W + i, pl.ds(off, 16)]
            return acc + dlt * dlt
        acc = lax.fori_loop(0, _NV, acc_body, jnp.zeros((16,), jnp.float32))
        s_i = jnp.sum(acc) * inv_t
        costv = costv + jnp.where(lane == i, s_i, 0.0)
    costref[...] = costv
    pltpu.sync_copy(costref, out_hbm.at[w])


def _sc_cost(px, py):
    mesh = plsc.VectorSubcoreMesh(core_axis_name="c", subcore_axis_name="s")
    fn = functools.partial(
        pl.kernel,
        mesh=mesh,
        out_type=jax.ShapeDtypeStruct((_NW, 16), jnp.float32),
        scratch_types=[
            pltpu.VMEM((2 * _PAIRS_PER_W, _T), jnp.float32),
            pltpu.VMEM((16,), jnp.float32),
        ],
        compiler_params=pltpu.CompilerParams(needs_layout_passes=False),
    )(_sc_cost_body)
    return fn(px, py)


def kernel(x, y, projections):
    px, py = _project(x, y, projections)
    cost = _sc_cost(px, py)
    cost_bl = cost[:, :_PAIRS_PER_W].reshape(_B, _NUM_PROJ)
    return jnp.clip(jnp.mean(cost_bl, axis=-1), _EPS, None) ** 0.5


# speedup vs baseline: 1309.2886x; 1.2439x over previous
import functools

import jax
import jax.numpy as jnp
from jax import lax
from jax.experimental import pallas as pl
from jax.experimental.pallas import tpu as pltpu
from jax.experimental.pallas import tpu_sc as plsc

_NUM_PROJ = 32
_EPS = 1e-06
_B, _T, _D = 4, 4096, 1024
_TB = 1024
_ROWS = _B * _NUM_PROJ
_NW = 32
_PAIRS_PER_W = _ROWS // _NW
_NV = _T // 16


def _dot_lt(a, b):
    return lax.dot_general(
        a, b, (((0,), (1,)), ((), ())),
        preferred_element_type=jnp.float32)


def _proj_body(x_ref, y_ref, p_ref, px_ref, py_ref):
    p = p_ref[...]
    pss = jnp.sum(p * p, axis=0, keepdims=True)
    pn = p / jnp.maximum(jnp.sqrt(pss), _EPS)
    pnh = pn.astype(jnp.bfloat16)
    pnl = (pn - pnh.astype(jnp.float32)).astype(jnp.bfloat16)
    for src, dst in ((x_ref, px_ref), (y_ref, py_ref)):
        v = src[0]
        ss = jnp.sum(v * v, axis=1)
        s = 1.0 / jnp.maximum(jnp.sqrt(ss), _EPS)
        vh = v.astype(jnp.bfloat16)
        vl = (v - vh.astype(jnp.float32)).astype(jnp.bfloat16)
        out = _dot_lt(pnh, vh) + (_dot_lt(pnl, vh) + _dot_lt(pnh, vl))
        dst[...] = out * s[None, :]


def _project(x, y, projections):
    grid = (_B, _T // _TB)
    return pl.pallas_call(
        _proj_body,
        grid=grid,
        in_specs=[
            pl.BlockSpec((1, _TB, _D), lambda b, t: (b, t, 0)),
            pl.BlockSpec((1, _TB, _D), lambda b, t: (b, t, 0)),
            pl.BlockSpec((_D, _NUM_PROJ), lambda b, t: (0, 0)),
        ],
        out_specs=[
            pl.BlockSpec((_NUM_PROJ, _TB), lambda b, t: (b, t)),
            pl.BlockSpec((_NUM_PROJ, _TB), lambda b, t: (b, t)),
        ],
        out_shape=[
            jax.ShapeDtypeStruct((_ROWS, _T), jnp.float32),
            jax.ShapeDtypeStruct((_ROWS, _T), jnp.float32),
        ],
    )(x, y, projections)


def _sort16_pass(buf):
    def body(j, c):
        off = j * 16
        for r in range(8):
            buf[r, pl.ds(off, 16)] = jnp.sort(buf[r, pl.ds(off, 16)])
        return c
    lax.fori_loop(0, _NV, body, 0)


def _sc_cost_body(px_hbm, py_hbm, out_hbm, buf, costref):
    c = lax.axis_index("c")
    s = lax.axis_index("s")
    w = s * 2 + c
    base = w * _PAIRS_PER_W
    pltpu.sync_copy(px_hbm.at[pl.ds(base, _PAIRS_PER_W)],
                    buf.at[pl.ds(0, _PAIRS_PER_W)])
    pltpu.sync_copy(py_hbm.at[pl.ds(base, _PAIRS_PER_W)],
                    buf.at[pl.ds(_PAIRS_PER_W, _PAIRS_PER_W)])

    _sort16_pass(buf)

    for n in (16, 32, 64, 128, 256, 512, 1024, 2048):
        nb = n // 16

        def cross_body(p, c, nb=nb, n=n):
            blk = p // nb
            r16 = p % nb
            ai = blk * (2 * n) + r16 * 16
            bi = blk * (2 * n) + 2 * n - r16 * 16 - 16
            for r in range(8):
                av = buf[r, pl.ds(ai, 16)]
                bv = jnp.flip(buf[r, pl.ds(bi, 16)], axis=0)
                buf[r, pl.ds(ai, 16)] = jnp.minimum(av, bv)
                buf[r, pl.ds(bi, 16)] = jnp.flip(jnp.maximum(av, bv), axis=0)
            return c
        lax.fori_loop(0, _T // 32, cross_body, 0)

        d = n // 2
        while d >= 16:
            q = d // 16

            def stage_body(p, c, q=q, d=d):
                blk = p // q
                r16 = p % q
                i0 = blk * (2 * d) + r16 * 16
                i1 = i0 + d
                for r in range(8):
                    av = buf[r, pl.ds(i0, 16)]
                    bv = buf[r, pl.ds(i1, 16)]
                    buf[r, pl.ds(i0, 16)] = jnp.minimum(av, bv)
                    buf[r, pl.ds(i1, 16)] = jnp.maximum(av, bv)
                return c
            lax.fori_loop(0, _T // 32, stage_body, 0)
            d //= 2

        _sort16_pass(buf)

    lane = lax.iota(jnp.int32, 16)
    costv = jnp.zeros((16,), jnp.float32)
    inv_t = 1.0 / _T
    for i in range(_PAIRS_PER_W):
        def acc_body(j, acc, i=i):
            off = j * 16
            dlt = buf[i, pl.ds(off, 16)] - buf[_PAIRS_PER_W + i, pl.ds(off, 16)]
            return acc + dlt * dlt
        acc = lax.fori_loop(0, _NV, acc_body, jnp.zeros((16,), jnp.float32))
        s_i = jnp.sum(acc) * inv_t
        costv = costv + jnp.where(lane == i, s_i, 0.0)
    costref[...] = costv
    pltpu.sync_copy(costref, out_hbm.at[w])


def _sc_cost(px, py):
    mesh = plsc.VectorSubcoreMesh(core_axis_name="c", subcore_axis_name="s")
    fn = functools.partial(
        pl.kernel,
        mesh=mesh,
        out_type=jax.ShapeDtypeStruct((_NW, 16), jnp.float32),
        scratch_types=[
            pltpu.VMEM((2 * _PAIRS_PER_W, _T), jnp.float32),
            pltpu.VMEM((16,), jnp.float32),
        ],
        compiler_params=pltpu.CompilerParams(needs_layout_passes=False),
    )(_sc_cost_body)
    return fn(px, py)


def kernel(x, y, projections):
    px, py = _project(x, y, projections)
    cost = _sc_cost(px, py)
    cost_bl = cost[:, :_PAIRS_PER_W].reshape(_B, _NUM_PROJ)
    return jnp.clip(jnp.mean(cost_bl, axis=-1), _EPS, None) ** 0.5


# speedup vs baseline: 2402.2363x; 1.8348x over previous
import functools

import jax
import jax.numpy as jnp
from jax import lax
from jax.experimental import pallas as pl
from jax.experimental.pallas import tpu as pltpu
from jax.experimental.pallas import tpu_sc as plsc

_NUM_PROJ = 32
_EPS = 1e-06
_B, _T, _D = 4, 4096, 1024
_TB = 1024
_ROWS = _B * _NUM_PROJ
_NW = 32
_PAIRS_PER_W = _ROWS // _NW
_NV = _T // 16


def _dot_lt(a, b):
    return lax.dot_general(
        a, b, (((0,), (1,)), ((), ())),
        preferred_element_type=jnp.float32)


def _proj_body(x_ref, y_ref, p_ref, px_ref, py_ref):
    p = p_ref[...]
    pss = jnp.sum(p * p, axis=0, keepdims=True)
    pn = p / jnp.maximum(jnp.sqrt(pss), _EPS)
    pnh = pn.astype(jnp.bfloat16)
    pnl = (pn - pnh.astype(jnp.float32)).astype(jnp.bfloat16)
    for src, dst in ((x_ref, px_ref), (y_ref, py_ref)):
        v = src[0]
        ss = jnp.sum(v * v, axis=1)
        s = 1.0 / jnp.maximum(jnp.sqrt(ss), _EPS)
        vh = v.astype(jnp.bfloat16)
        vl = (v - vh.astype(jnp.float32)).astype(jnp.bfloat16)
        out = _dot_lt(pnh, vh) + (_dot_lt(pnl, vh) + _dot_lt(pnh, vl))
        dst[...] = out * s[None, :]


def _project(x, y, projections):
    grid = (_B, _T // _TB)
    return pl.pallas_call(
        _proj_body,
        grid=grid,
        in_specs=[
            pl.BlockSpec((1, _TB, _D), lambda b, t: (b, t, 0)),
            pl.BlockSpec((1, _TB, _D), lambda b, t: (b, t, 0)),
            pl.BlockSpec((_D, _NUM_PROJ), lambda b, t: (0, 0)),
        ],
        out_specs=[
            pl.BlockSpec((_NUM_PROJ, _TB), lambda b, t: (b, t)),
            pl.BlockSpec((_NUM_PROJ, _TB), lambda b, t: (b, t)),
        ],
        out_shape=[
            jax.ShapeDtypeStruct((_ROWS, _T), jnp.float32),
            jax.ShapeDtypeStruct((_ROWS, _T), jnp.float32),
        ],
    )(x, y, projections)


_NPAIR = _T // 32


def _sc_cost_body(px_hbm, py_hbm, out_hbm, buf, costref):
    c = lax.axis_index("c")
    s = lax.axis_index("s")
    w = s * 2 + c
    base = w * _PAIRS_PER_W
    pltpu.sync_copy(px_hbm.at[pl.ds(base, _PAIRS_PER_W)],
                    buf.at[pl.ds(0, _PAIRS_PER_W)])
    pltpu.sync_copy(py_hbm.at[pl.ds(base, _PAIRS_PER_W)],
                    buf.at[pl.ds(_PAIRS_PER_W, _PAIRS_PER_W)])

    @plsc.parallel_loop(0, _NPAIR, 1, unroll=1)
    def _lvl16(pp):
        ai = pp * 32
        bi = ai + 16
        for r in range(8):
            a = jnp.sort(buf[r, pl.ds(ai, 16)])
            b = jnp.sort(buf[r, pl.ds(bi, 16)])
            bv = jnp.flip(b, axis=0)
            buf[r, pl.ds(ai, 16)] = jnp.sort(jnp.minimum(a, bv))
            buf[r, pl.ds(bi, 16)] = jnp.sort(jnp.maximum(a, bv))

    for n in (32, 64, 128, 256, 512, 1024, 2048):
        nb = n // 16

        @plsc.parallel_loop(0, _NPAIR, 1, unroll=1)
        def _cross(p, nb=nb, n=n):
            blk = p // nb
            r16 = p % nb
            ai = blk * (2 * n) + r16 * 16
            bi = blk * (2 * n) + 2 * n - r16 * 16 - 16
            for r in range(8):
                av = buf[r, pl.ds(ai, 16)]
                bv = jnp.flip(buf[r, pl.ds(bi, 16)], axis=0)
                buf[r, pl.ds(ai, 16)] = jnp.minimum(av, bv)
                buf[r, pl.ds(bi, 16)] = jnp.flip(jnp.maximum(av, bv), axis=0)

        d = n // 2
        while d >= 32:
            q = d // 16

            @plsc.parallel_loop(0, _NPAIR, 1, unroll=1)
            def _stage(p, q=q, d=d):
                blk = p // q
                r16 = p % q
                i0 = blk * (2 * d) + r16 * 16
                i1 = i0 + d
                for r in range(8):
                    av = buf[r, pl.ds(i0, 16)]
                    bv = buf[r, pl.ds(i1, 16)]
                    buf[r, pl.ds(i0, 16)] = jnp.minimum(av, bv)
                    buf[r, pl.ds(i1, 16)] = jnp.maximum(av, bv)
            d //= 2

        @plsc.parallel_loop(0, _NPAIR, 1, unroll=1)
        def _finish(pp):
            i0 = pp * 32
            i1 = i0 + 16
            for r in range(8):
                av = buf[r, pl.ds(i0, 16)]
                bv = buf[r, pl.ds(i1, 16)]
                buf[r, pl.ds(i0, 16)] = jnp.sort(jnp.minimum(av, bv))
                buf[r, pl.ds(i1, 16)] = jnp.sort(jnp.maximum(av, bv))

    lane = lax.iota(jnp.int32, 16)
    costv = jnp.zeros((16,), jnp.float32)
    inv_t = 1.0 / _T
    for i in range(_PAIRS_PER_W):
        def acc_body(j, acc, i=i):
            off = j * 16
            dlt = buf[i, pl.ds(off, 16)] - buf[_PAIRS_PER_W + i, pl.ds(off, 16)]
            return acc + dlt * dlt
        acc = lax.fori_loop(0, _NV, acc_body, jnp.zeros((16,), jnp.float32))
        s_i = jnp.sum(acc) * inv_t
        costv = costv + jnp.where(lane == i, s_i, 0.0)
    costref[...] = costv
    pltpu.sync_copy(costref, out_hbm.at[w])


def _sc_cost(px, py):
    mesh = plsc.VectorSubcoreMesh(core_axis_name="c", subcore_axis_name="s")
    fn = functools.partial(
        pl.kernel,
        mesh=mesh,
        out_type=jax.ShapeDtypeStruct((_NW, 16), jnp.float32),
        scratch_types=[
            pltpu.VMEM((2 * _PAIRS_PER_W, _T), jnp.float32),
            pltpu.VMEM((16,), jnp.float32),
        ],
        compiler_params=pltpu.CompilerParams(needs_layout_passes=False),
    )(_sc_cost_body)
    return fn(px, py)


def kernel(x, y, projections):
    px, py = _project(x, y, projections)
    cost = _sc_cost(px, py)
    cost_bl = cost[:, :_PAIRS_PER_W].reshape(_B, _NUM_PROJ)
    return jnp.clip(jnp.mean(cost_bl, axis=-1), _EPS, None) ** 0.5


# speedup vs baseline: 2410.6481x; 1.0035x over previous
import functools

import jax
import jax.numpy as jnp
from jax import lax
from jax.experimental import pallas as pl
from jax.experimental.pallas import tpu as pltpu
from jax.experimental.pallas import tpu_sc as plsc

_NUM_PROJ = 32
_EPS = 1e-06
_B, _T, _D = 4, 4096, 1024
_TB = 2048
_ROWS = _B * _NUM_PROJ
_NW = 32
_PAIRS_PER_W = _ROWS // _NW
_NV = _T // 16


def _dot_lt(a, b):
    return lax.dot_general(
        a, b, (((0,), (1,)), ((), ())),
        preferred_element_type=jnp.float32)


def _proj_body(x_ref, y_ref, p_ref, px_ref, py_ref):
    p = p_ref[...]
    pss = jnp.sum(p * p, axis=0, keepdims=True)
    pn = p / jnp.maximum(jnp.sqrt(pss), _EPS)
    pnh = pn.astype(jnp.bfloat16)
    pnl = (pn - pnh.astype(jnp.float32)).astype(jnp.bfloat16)
    for src, dst in ((x_ref, px_ref), (y_ref, py_ref)):
        v = src[0]
        ss = jnp.sum(v * v, axis=1)
        s = 1.0 / jnp.maximum(jnp.sqrt(ss), _EPS)
        vh = v.astype(jnp.bfloat16)
        vl = (v - vh.astype(jnp.float32)).astype(jnp.bfloat16)
        out = _dot_lt(pnh, vh) + (_dot_lt(pnl, vh) + _dot_lt(pnh, vl))
        dst[...] = out * s[None, :]


def _project(x, y, projections):
    grid = (_B, _T // _TB)
    return pl.pallas_call(
        _proj_body,
        grid=grid,
        in_specs=[
            pl.BlockSpec((1, _TB, _D), lambda b, t: (b, t, 0)),
            pl.BlockSpec((1, _TB, _D), lambda b, t: (b, t, 0)),
            pl.BlockSpec((_D, _NUM_PROJ), lambda b, t: (0, 0)),
        ],
        out_specs=[
            pl.BlockSpec((_NUM_PROJ, _TB), lambda b, t: (b, t)),
            pl.BlockSpec((_NUM_PROJ, _TB), lambda b, t: (b, t)),
        ],
        out_shape=[
            jax.ShapeDtypeStruct((_ROWS, _T), jnp.float32),
            jax.ShapeDtypeStruct((_ROWS, _T), jnp.float32),
        ],
    )(x, y, projections)


_NPAIR = _T // 32


def _sc_cost_body(px_hbm, py_hbm, out_hbm, buf, costref):
    c = lax.axis_index("c")
    s = lax.axis_index("s")
    w = s * 2 + c
    base = w * _PAIRS_PER_W
    pltpu.sync_copy(px_hbm.at[pl.ds(base, _PAIRS_PER_W)],
                    buf.at[pl.ds(0, _PAIRS_PER_W)])
    pltpu.sync_copy(py_hbm.at[pl.ds(base, _PAIRS_PER_W)],
                    buf.at[pl.ds(_PAIRS_PER_W, _PAIRS_PER_W)])

    @plsc.parallel_loop(0, _NPAIR, 1, unroll=1)
    def _lvl16(pp):
        ai = pp * 32
        bi = ai + 16
        for r in range(8):
            a = jnp.sort(buf[r, pl.ds(ai, 16)])
            b = jnp.sort(buf[r, pl.ds(bi, 16)])
            bv = jnp.flip(b, axis=0)
            buf[r, pl.ds(ai, 16)] = jnp.sort(jnp.minimum(a, bv))
            buf[r, pl.ds(bi, 16)] = jnp.sort(jnp.maximum(a, bv))

    for n in (32, 64, 128, 256, 512, 1024, 2048):
        nb = n // 16

        @plsc.parallel_loop(0, _NPAIR, 1, unroll=1)
        def _cross(p, nb=nb, n=n):
            blk = p // nb
            r16 = p % nb
            ai = blk * (2 * n) + r16 * 16
            bi = blk * (2 * n) + 2 * n - r16 * 16 - 16
            for r in range(8):
                av = buf[r, pl.ds(ai, 16)]
                bv = jnp.flip(buf[r, pl.ds(bi, 16)], axis=0)
                buf[r, pl.ds(ai, 16)] = jnp.minimum(av, bv)
                buf[r, pl.ds(bi, 16)] = jnp.flip(jnp.maximum(av, bv), axis=0)

        d = n // 2
        while d >= 32:
            q = d // 16

            @plsc.parallel_loop(0, _NPAIR, 1, unroll=1)
            def _stage(p, q=q, d=d):
                blk = p // q
                r16 = p % q
                i0 = blk * (2 * d) + r16 * 16
                i1 = i0 + d
                for r in range(8):
                    av = buf[r, pl.ds(i0, 16)]
                    bv = buf[r, pl.ds(i1, 16)]
                    buf[r, pl.ds(i0, 16)] = jnp.minimum(av, bv)
                    buf[r, pl.ds(i1, 16)] = jnp.maximum(av, bv)
            d //= 2

        @plsc.parallel_loop(0, _NPAIR, 1, unroll=1)
        def _finish(pp):
            i0 = pp * 32
            i1 = i0 + 16
            for r in range(8):
                av = buf[r, pl.ds(i0, 16)]
                bv = buf[r, pl.ds(i1, 16)]
                buf[r, pl.ds(i0, 16)] = jnp.sort(jnp.minimum(av, bv))
                buf[r, pl.ds(i1, 16)] = jnp.sort(jnp.maximum(av, bv))

    lane = lax.iota(jnp.int32, 16)
    costv = jnp.zeros((16,), jnp.float32)
    inv_t = 1.0 / _T
    for i in range(_PAIRS_PER_W):
        def acc_body(j, acc, i=i):
            off = j * 16
            dlt = buf[i, pl.ds(off, 16)] - buf[_PAIRS_PER_W + i, pl.ds(off, 16)]
            return acc + dlt * dlt
        acc = lax.fori_loop(0, _NV, acc_body, jnp.zeros((16,), jnp.float32))
        s_i = jnp.sum(acc) * inv_t
        costv = costv + jnp.where(lane == i, s_i, 0.0)
    costref[...] = costv
    pltpu.sync_copy(costref, out_hbm.at[w])


def _sc_cost(px, py):
    mesh = plsc.VectorSubcoreMesh(core_axis_name="c", subcore_axis_name="s")
    fn = functools.partial(
        pl.kernel,
        mesh=mesh,
        out_type=jax.ShapeDtypeStruct((_NW, 16), jnp.float32),
        scratch_types=[
            pltpu.VMEM((2 * _PAIRS_PER_W, _T), jnp.float32),
            pltpu.VMEM((16,), jnp.float32),
        ],
        compiler_params=pltpu.CompilerParams(needs_layout_passes=False),
    )(_sc_cost_body)
    return fn(px, py)


def kernel(x, y, projections):
    px, py = _project(x, y, projections)
    cost = _sc_cost(px, py)
    cost_bl = cost[:, :_PAIRS_PER_W].reshape(_B, _NUM_PROJ)
    return jnp.clip(jnp.mean(cost_bl, axis=-1), _EPS, None) ** 0.5


# speedup vs baseline: 2680.2611x; 1.1118x over previous
import functools

import jax
import jax.numpy as jnp
from jax import lax
from jax.experimental import pallas as pl
from jax.experimental.pallas import tpu as pltpu
from jax.experimental.pallas import tpu_sc as plsc

_NUM_PROJ = 32
_EPS = 1e-06
_B, _T, _D = 4, 4096, 1024
_TB = 2048
_NW = 32
_NV = _T // 16
_NPAIR = _T // 32
_NCHUNK = 2
_BC = _B // _NCHUNK
_CROWS = _BC * _NUM_PROJ
_PPW = _CROWS // _NW
_LSTEP = 2 * _PPW


def _dot_lt(a, b):
    return lax.dot_general(
        a, b, (((0,), (1,)), ((), ())),
        preferred_element_type=jnp.float32)


def _proj_body(x_ref, y_ref, p_ref, px_ref, py_ref):
    p = p_ref[...]
    pss = jnp.sum(p * p, axis=0, keepdims=True)
    pn = p / jnp.maximum(jnp.sqrt(pss), _EPS)
    pnh = pn.astype(jnp.bfloat16)
    pnl = (pn - pnh.astype(jnp.float32)).astype(jnp.bfloat16)
    for src, dst in ((x_ref, px_ref), (y_ref, py_ref)):
        v = src[0]
        ss = jnp.sum(v * v, axis=1)
        s = 1.0 / jnp.maximum(jnp.sqrt(ss), _EPS)
        vh = v.astype(jnp.bfloat16)
        vl = (v - vh.astype(jnp.float32)).astype(jnp.bfloat16)
        out = _dot_lt(pnh, vh) + (_dot_lt(pnl, vh) + _dot_lt(pnh, vl))
        dst[...] = out * s[None, :]


def _project(x, y, projections, chunk):
    grid = (_BC, _T // _TB)
    boff = chunk * _BC
    return pl.pallas_call(
        _proj_body,
        grid=grid,
        in_specs=[
            pl.BlockSpec((1, _TB, _D), lambda b, t: (b + boff, t, 0)),
            pl.BlockSpec((1, _TB, _D), lambda b, t: (b + boff, t, 0)),
            pl.BlockSpec((_D, _NUM_PROJ), lambda b, t: (0, 0)),
        ],
        out_specs=[
            pl.BlockSpec((_NUM_PROJ, _TB), lambda b, t: (b, t)),
            pl.BlockSpec((_NUM_PROJ, _TB), lambda b, t: (b, t)),
        ],
        out_shape=[
            jax.ShapeDtypeStruct((_CROWS, _T), jnp.float32),
            jax.ShapeDtypeStruct((_CROWS, _T), jnp.float32),
        ],
    )(x, y, projections)


def _sc_cost_body(px_hbm, py_hbm, out_hbm, buf, costref):
    c = lax.axis_index("c")
    s = lax.axis_index("s")
    w = s * 2 + c
    base = w * _PPW
    pltpu.sync_copy(px_hbm.at[pl.ds(base, _PPW)], buf.at[pl.ds(0, _PPW)])
    pltpu.sync_copy(py_hbm.at[pl.ds(base, _PPW)], buf.at[pl.ds(_PPW, _PPW)])

    @plsc.parallel_loop(0, _NPAIR, 1, unroll=1)
    def _lvl16(pp):
        ai = pp * 32
        bi = ai + 16
        for r in range(_LSTEP):
            a = jnp.sort(buf[r, pl.ds(ai, 16)])
            b = jnp.sort(buf[r, pl.ds(bi, 16)])
            bv = jnp.flip(b, axis=0)
            buf[r, pl.ds(ai, 16)] = jnp.sort(jnp.minimum(a, bv))
            buf[r, pl.ds(bi, 16)] = jnp.sort(jnp.maximum(a, bv))

    for n in (32, 64, 128, 256, 512, 1024, 2048):
        nb = n // 16

        @plsc.parallel_loop(0, _NPAIR, 1, unroll=1)
        def _cross(p, nb=nb, n=n):
            blk = p // nb
            r16 = p % nb
            ai = blk * (2 * n) + r16 * 16
            bi = blk * (2 * n) + 2 * n - r16 * 16 - 16
            for r in range(_LSTEP):
                av = buf[r, pl.ds(ai, 16)]
                bv = jnp.flip(buf[r, pl.ds(bi, 16)], axis=0)
                buf[r, pl.ds(ai, 16)] = jnp.minimum(av, bv)
                buf[r, pl.ds(bi, 16)] = jnp.flip(jnp.maximum(av, bv), axis=0)

        d = n // 2
        while d >= 32:
            q = d // 16

            @plsc.parallel_loop(0, _NPAIR, 1, unroll=1)
            def _stage(p, q=q, d=d):
                blk = p // q
                r16 = p % q
                i0 = blk * (2 * d) + r16 * 16
                i1 = i0 + d
                for r in range(_LSTEP):
                    av = buf[r, pl.ds(i0, 16)]
                    bv = buf[r, pl.ds(i1, 16)]
                    buf[r, pl.ds(i0, 16)] = jnp.minimum(av, bv)
                    buf[r, pl.ds(i1, 16)] = jnp.maximum(av, bv)
            d //= 2

        @plsc.parallel_loop(0, _NPAIR, 1, unroll=1)
        def _finish(pp):
            i0 = pp * 32
            i1 = i0 + 16
            for r in range(_LSTEP):
                av = buf[r, pl.ds(i0, 16)]
                bv = buf[r, pl.ds(i1, 16)]
                buf[r, pl.ds(i0, 16)] = jnp.sort(jnp.minimum(av, bv))
                buf[r, pl.ds(i1, 16)] = jnp.sort(jnp.maximum(av, bv))

    lane = lax.iota(jnp.int32, 16)
    costv = jnp.zeros((16,), jnp.float32)
    inv_t = 1.0 / _T
    for i in range(_PPW):
        def acc_body(j, acc, i=i):
            off = j * 16
            dlt = buf[i, pl.ds(off, 16)] - buf[_PPW + i, pl.ds(off, 16)]
            return acc + dlt * dlt
        acc = lax.fori_loop(0, _NV, acc_body, jnp.zeros((16,), jnp.float32))
        s_i = jnp.sum(acc) * inv_t
        costv = costv + jnp.where(lane == i, s_i, 0.0)
    costref[...] = costv
    pltpu.sync_copy(costref, out_hbm.at[w])


def _sc_cost(px, py):
    mesh = plsc.VectorSubcoreMesh(core_axis_name="c", subcore_axis_name="s")
    fn = functools.partial(
        pl.kernel,
        mesh=mesh,
        out_type=jax.ShapeDtypeStruct((_NW, 16), jnp.float32),
        scratch_types=[
            pltpu.VMEM((_LSTEP, _T), jnp.float32),
            pltpu.VMEM((16,), jnp.float32),
        ],
        compiler_params=pltpu.CompilerParams(needs_layout_passes=False),
    )(_sc_cost_body)
    return fn(px, py)


def kernel(x, y, projections):
    costs = []
    for chunk in range(_NCHUNK):
        px, py = _project(x, y, projections, chunk)
        cost = _sc_cost(px, py)
        costs.append(cost[:, :_PPW].reshape(_BC, _NUM_PROJ))
    cost_bl = jnp.concatenate(costs, axis=0)
    return jnp.clip(jnp.mean(cost_bl, axis=-1), _EPS, None) ** 0.5


# speedup vs baseline: 2981.8423x; 1.1125x over previous
import functools

import jax
import jax.numpy as jnp
from jax import lax
from jax.experimental import pallas as pl
from jax.experimental.pallas import tpu as pltpu
from jax.experimental.pallas import tpu_sc as plsc

_NUM_PROJ = 32
_EPS = 1e-06
_B, _T, _D = 4, 4096, 1024
_TB = 2048
_NW = 32
_NV = _T // 16
_NPAIR = _T // 32
_NCHUNK = 2
_BC = _B // _NCHUNK
_CROWS = _BC * _NUM_PROJ
_PPW = _CROWS // _NW
_LSTEP = 2 * _PPW


def _dot_lt(a, b):
    return lax.dot_general(
        a, b, (((0,), (1,)), ((), ())),
        preferred_element_type=jnp.float32)


def _proj_body(x_ref, y_ref, p_ref, px_ref, py_ref):
    p = p_ref[...]
    pss = jnp.sum(p * p, axis=0, keepdims=True)
    pn = p / jnp.maximum(jnp.sqrt(pss), _EPS)
    pnh = pn.astype(jnp.bfloat16)
    for src, dst in ((x_ref, px_ref), (y_ref, py_ref)):
        v = src[0]
        ss = jnp.sum(v * v, axis=1)
        s = 1.0 / jnp.maximum(jnp.sqrt(ss), _EPS)
        out = _dot_lt(pnh, v.astype(jnp.bfloat16))
        dst[...] = out * s[None, :]


def _project(x, y, projections, chunk):
    grid = (_BC, _T // _TB)
    boff = chunk * _BC
    return pl.pallas_call(
        _proj_body,
        grid=grid,
        in_specs=[
            pl.BlockSpec((1, _TB, _D), lambda b, t: (b + boff, t, 0)),
            pl.BlockSpec((1, _TB, _D), lambda b, t: (b + boff, t, 0)),
            pl.BlockSpec((_D, _NUM_PROJ), lambda b, t: (0, 0)),
        ],
        out_specs=[
            pl.BlockSpec((_NUM_PROJ, _TB), lambda b, t: (b, t)),
            pl.BlockSpec((_NUM_PROJ, _TB), lambda b, t: (b, t)),
        ],
        out_shape=[
            jax.ShapeDtypeStruct((_CROWS, _T), jnp.float32),
            jax.ShapeDtypeStruct((_CROWS, _T), jnp.float32),
        ],
    )(x, y, projections)


def _sc_cost_body(px_hbm, py_hbm, out_hbm, buf, costref):
    c = lax.axis_index("c")
    s = lax.axis_index("s")
    w = s * 2 + c
    base = w * _PPW
    pltpu.sync_copy(px_hbm.at[pl.ds(base, _PPW)], buf.at[pl.ds(0, _PPW)])
    pltpu.sync_copy(py_hbm.at[pl.ds(base, _PPW)], buf.at[pl.ds(_PPW, _PPW)])

    @plsc.parallel_loop(0, _NPAIR, 1, unroll=1)
    def _lvl16(pp):
        ai = pp * 32
        bi = ai + 16
        for r in range(_LSTEP):
            a = jnp.sort(buf[r, pl.ds(ai, 16)])
            b = jnp.sort(buf[r, pl.ds(bi, 16)])
            bv = jnp.flip(b, axis=0)
            buf[r, pl.ds(ai, 16)] = jnp.sort(jnp.minimum(a, bv))
            buf[r, pl.ds(bi, 16)] = jnp.sort(jnp.maximum(a, bv))

    for n in (32, 64, 128, 256, 512, 1024, 2048):
        nb = n // 16

        @plsc.parallel_loop(0, _NPAIR, 1, unroll=1)
        def _cross(p, nb=nb, n=n):
            blk = p // nb
            r16 = p % nb
            ai = blk * (2 * n) + r16 * 16
            bi = blk * (2 * n) + 2 * n - r16 * 16 - 16
            for r in range(_LSTEP):
                av = buf[r, pl.ds(ai, 16)]
                bv = jnp.flip(buf[r, pl.ds(bi, 16)], axis=0)
                buf[r, pl.ds(ai, 16)] = jnp.minimum(av, bv)
                buf[r, pl.ds(bi, 16)] = jnp.flip(jnp.maximum(av, bv), axis=0)

        d = n // 2
        while d >= 32:
            q = d // 16

            @plsc.parallel_loop(0, _NPAIR, 1, unroll=1)
            def _stage(p, q=q, d=d):
                blk = p // q
                r16 = p % q
                i0 = blk * (2 * d) + r16 * 16
                i1 = i0 + d
                for r in range(_LSTEP):
                    av = buf[r, pl.ds(i0, 16)]
                    bv = buf[r, pl.ds(i1, 16)]
                    buf[r, pl.ds(i0, 16)] = jnp.minimum(av, bv)
                    buf[r, pl.ds(i1, 16)] = jnp.maximum(av, bv)
            d //= 2

        @plsc.parallel_loop(0, _NPAIR, 1, unroll=1)
        def _finish(pp):
            i0 = pp * 32
            i1 = i0 + 16
            for r in range(_LSTEP):
                av = buf[r, pl.ds(i0, 16)]
                bv = buf[r, pl.ds(i1, 16)]
                buf[r, pl.ds(i0, 16)] = jnp.sort(jnp.minimum(av, bv))
                buf[r, pl.ds(i1, 16)] = jnp.sort(jnp.maximum(av, bv))

    lane = lax.iota(jnp.int32, 16)
    costv = jnp.zeros((16,), jnp.float32)
    inv_t = 1.0 / _T
    for i in range(_PPW):
        def acc_body(j, acc, i=i):
            off = j * 16
            dlt = buf[i, pl.ds(off, 16)] - buf[_PPW + i, pl.ds(off, 16)]
            return acc + dlt * dlt
        acc = lax.fori_loop(0, _NV, acc_body, jnp.zeros((16,), jnp.float32))
        s_i = jnp.sum(acc) * inv_t
        costv = costv + jnp.where(lane == i, s_i, 0.0)
    costref[...] = costv
    pltpu.sync_copy(costref, out_hbm.at[w])


def _sc_cost(px, py):
    mesh = plsc.VectorSubcoreMesh(core_axis_name="c", subcore_axis_name="s")
    fn = functools.partial(
        pl.kernel,
        mesh=mesh,
        out_type=jax.ShapeDtypeStruct((_NW, 16), jnp.float32),
        scratch_types=[
            pltpu.VMEM((_LSTEP, _T), jnp.float32),
            pltpu.VMEM((16,), jnp.float32),
        ],
        compiler_params=pltpu.CompilerParams(needs_layout_passes=False),
    )(_sc_cost_body)
    return fn(px, py)


def kernel(x, y, projections):
    costs = []
    for chunk in range(_NCHUNK):
        px, py = _project(x, y, projections, chunk)
        cost = _sc_cost(px, py)
        costs.append(cost[:, :_PPW].reshape(_BC, _NUM_PROJ))
    cost_bl = jnp.concatenate(costs, axis=0)
    return jnp.clip(jnp.mean(cost_bl, axis=-1), _EPS, None) ** 0.5
